# baseline (speedup 1.0000x reference)
import functools

import jax
import jax.numpy as jnp
from jax import lax
from jax.experimental import pallas as pl
from jax.experimental.pallas import tpu as pltpu
from jax.experimental.pallas import tpu_sc as plsc

N_NODES = 10000
N_EDGES = 320000
D_FEAT = 128
D_EDGE = 16

NC = 2
NS = 16
NW = NC * NS
G = 128
N_CHUNKS = N_EDGES // G
NCW = N_CHUNKS // NW
N_EXTRA = N_CHUNKS % NW
CHUNK_F32 = G * D_EDGE
HALF = CHUNK_F32 // 2
BAND_F32 = N_CHUNKS * 8 * G
NSLOT = 6
NTRI = NCW // NSLOT


def _tc_tables(x_ref, w_ref, b_ref, p_ref, q_ref):
    res = jnp.dot(x_ref[...], w_ref[...], preferred_element_type=jnp.float32)
    p_ref[...] = res[:, :D_EDGE]
    q_ref[...] = res[:, D_EDGE:] + b_ref[...]


_mesh = plsc.VectorSubcoreMesh(core_axis_name="c", subcore_axis_name="s")


@functools.partial(
    pl.kernel,
    mesh=_mesh,
    compiler_params=pltpu.CompilerParams(use_tc_tiling_on_sc=False,
                                         needs_layout_passes=False),
    out_type=jax.ShapeDtypeStruct((N_EDGES * D_EDGE,), jnp.float32),
    scratch_types=[
        pltpu.VMEM((NCW, 2, G), jnp.int32),
        pltpu.VMEM((NSLOT, G, D_EDGE), jnp.float32),
        pltpu.VMEM((NSLOT, G, D_EDGE), jnp.float32),
    ] + [pltpu.VMEM((CHUNK_F32,), jnp.float32)] * NSLOT
      + [pltpu.SemaphoreType.DMA]
      + [pltpu.SemaphoreType.DMA] * (2 * NSLOT),
)
def _sc_gather_add(p_hbm, q_hbm, ei_hbm, out_hbm,
                   eibuf, pbuf, qbuf, *rest):
    obuf = rest[:NSLOT]
    sem_i = rest[NSLOT]
    sem_g = rest[NSLOT + 1:2 * NSLOT + 1]
    sem_o = rest[2 * NSLOT + 1:]
    wid = lax.axis_index("s") * NC + lax.axis_index("c")
    start_c = NCW * wid + jnp.minimum(wid, N_EXTRA)
    iot = lax.iota(jnp.int32, 16) * G

    pltpu.async_copy(ei_hbm.at[pl.ds(start_c, NCW)], eibuf, sem_i).wait()

    def fire_gathers(g, s):
        pltpu.async_copy(p_hbm.at[eibuf.at[g, 0]], pbuf.at[s], sem_g[s])
        pltpu.async_copy(q_hbm.at[eibuf.at[g, 1]], qbuf.at[s], sem_g[s])

    def wait_gathers(s):
        pltpu.make_async_copy(p_hbm.at[eibuf.at[0, 0]],
                              pbuf.at[s], sem_g[s]).wait()
        pltpu.make_async_copy(q_hbm.at[eibuf.at[0, 1]],
                              qbuf.at[s], sem_g[s]).wait()

    def add_rows(s):
        @plsc.parallel_loop(0, G, unroll=8)
        def row(i):
            val = pbuf[s, i, :] + qbuf[s, i, :]
            plsc.store_scatter(obuf[s], [iot + i], val)

    def fire_store_at(c, s, sem):
        pltpu.async_copy(obuf[s].at[pl.ds(0, HALF)],
                         out_hbm.at[pl.ds(c * HALF, HALF)], sem)
        pltpu.async_copy(obuf[s].at[pl.ds(HALF, HALF)],
                         out_hbm.at[pl.ds(BAND_F32 + c * HALF, HALF)], sem)

    def wait_store(s):
        pltpu.make_async_copy(obuf[s].at[pl.ds(0, HALF)],
                              out_hbm.at[pl.ds(0, HALF)], sem_o[s]).wait()
        pltpu.make_async_copy(obuf[s].at[pl.ds(0, HALF)],
                              out_hbm.at[pl.ds(0, HALF)], sem_o[s]).wait()

    def chunk(c, s, first, fire_next):
        wait_gathers(s)
        if not first:
            wait_store(s)
        add_rows(s)
        fire_store_at(start_c + c, s, sem_o[s])
        if fire_next:
            fire_gathers(c + NSLOT, s)

    for s in range(NSLOT):
        fire_gathers(s, s)
    for j in range(NSLOT):
        chunk(j, j, first=True, fire_next=True)

    def body(m, cy):
        for j in range(NSLOT):
            chunk(NSLOT * m + j, j, first=False, fire_next=True)
        return cy

    lax.fori_loop(1, NTRI - 1, body, 0)
    for j in range(NSLOT):
        chunk(NSLOT * (NTRI - 1) + j, j, first=False, fire_next=False)
    for s in range(NSLOT):
        wait_store(s)

    @pl.when(wid < N_EXTRA)
    def _extra():
        ec = start_c + NCW
        pltpu.sync_copy(ei_hbm.at[pl.ds(ec, 1)], eibuf.at[pl.ds(0, 1)])
        fire_gathers(0, 0)
        wait_gathers(0)
        add_rows(0)
        fire_store_at(ec, 0, sem_o[0])
        wait_store(0)


def kernel(x, edge_index, pos, W, b):
    wcat = jnp.concatenate([W[:D_FEAT, :], W[D_FEAT:, :]], axis=1)
    p, q = pl.pallas_call(
        _tc_tables,
        out_shape=[
            jax.ShapeDtypeStruct((N_NODES, D_EDGE), jnp.float32),
            jax.ShapeDtypeStruct((N_NODES, D_EDGE), jnp.float32),
        ],
    )(x, wcat, b.reshape(1, D_EDGE))
    ei3 = edge_index.reshape(2, N_CHUNKS, G).transpose(1, 0, 2)
    flat = _sc_gather_add(p, q, ei3)
    arr = flat.reshape(2, N_CHUNKS, 8, G)
    return arr.transpose(1, 3, 0, 2).reshape(N_EDGES, D_EDGE)


# speedup vs baseline: 1.0100x; 1.0100x over previous
import functools

import jax
import jax.numpy as jnp
from jax import lax
from jax.experimental import pallas as pl
from jax.experimental.pallas import tpu as pltpu
from jax.experimental.pallas import tpu_sc as plsc

N_NODES = 10000
N_EDGES = 320000
D_FEAT = 128
D_EDGE = 16

NC = 2
NS = 16
NW = NC * NS
G = 128
N_CHUNKS = N_EDGES // G
NCW = N_CHUNKS // NW
N_EXTRA = N_CHUNKS % NW
CHUNK_F32 = G * D_EDGE
HALF = CHUNK_F32 // 2
BAND_F32 = N_CHUNKS * 8 * G
NSLOT = 3
NTRI = NCW // NSLOT


def _tc_tables(x_ref, w_ref, b_ref, p_ref, q_ref):
    res = jnp.dot(x_ref[...], w_ref[...], preferred_element_type=jnp.float32)
    p_ref[...] = res[:, :D_EDGE]
    q_ref[...] = res[:, D_EDGE:] + b_ref[...]


_mesh = plsc.VectorSubcoreMesh(core_axis_name="c", subcore_axis_name="s")


@functools.partial(
    pl.kernel,
    mesh=_mesh,
    compiler_params=pltpu.CompilerParams(use_tc_tiling_on_sc=False,
                                         needs_layout_passes=False),
    out_type=jax.ShapeDtypeStruct((N_EDGES * D_EDGE,), jnp.float32),
    scratch_types=[
        pltpu.VMEM((NCW, 2, G), jnp.int32),
        pltpu.VMEM((NSLOT, G, D_EDGE), jnp.float32),
        pltpu.VMEM((NSLOT, G, D_EDGE), jnp.float32),
    ] + [pltpu.VMEM((CHUNK_F32,), jnp.float32)] * NSLOT
      + [pltpu.SemaphoreType.DMA]
      + [pltpu.SemaphoreType.DMA] * (2 * NSLOT),
)
def _sc_gather_add(p_hbm, q_hbm, ei_hbm, out_hbm,
                   eibuf, pbuf, qbuf, *rest):
    obuf = rest[:NSLOT]
    sem_i = rest[NSLOT]
    sem_g = rest[NSLOT + 1:2 * NSLOT + 1]
    sem_o = rest[2 * NSLOT + 1:]
    wid = lax.axis_index("s") * NC + lax.axis_index("c")
    start_c = NCW * wid + jnp.minimum(wid, N_EXTRA)
    iot = lax.iota(jnp.int32, 16) * G

    pltpu.async_copy(ei_hbm.at[pl.ds(start_c, NCW)], eibuf, sem_i).wait()

    def fire_gathers(g, s):
        pltpu.async_copy(p_hbm.at[eibuf.at[g, 0]], pbuf.at[s], sem_g[s])
        pltpu.async_copy(q_hbm.at[eibuf.at[g, 1]], qbuf.at[s], sem_g[s])

    def wait_gathers(s):
        pltpu.make_async_copy(p_hbm.at[eibuf.at[0, 0]],
                              pbuf.at[s], sem_g[s]).wait()
        pltpu.make_async_copy(q_hbm.at[eibuf.at[0, 1]],
                              qbuf.at[s], sem_g[s]).wait()

    def add_rows(s):
        @plsc.parallel_loop(0, G, unroll=8)
        def row(i):
            val = pbuf[s, i, :] + qbuf[s, i, :]
            plsc.store_scatter(obuf[s], [iot + i], val)

    def fire_store_at(c, s, sem):
        pltpu.async_copy(obuf[s].at[pl.ds(0, HALF)],
                         out_hbm.at[pl.ds(c * HALF, HALF)], sem)
        pltpu.async_copy(obuf[s].at[pl.ds(HALF, HALF)],
                         out_hbm.at[pl.ds(BAND_F32 + c * HALF, HALF)], sem)

    def wait_store(s):
        pltpu.make_async_copy(obuf[s].at[pl.ds(0, HALF)],
                              out_hbm.at[pl.ds(0, HALF)], sem_o[s]).wait()
        pltpu.make_async_copy(obuf[s].at[pl.ds(0, HALF)],
                              out_hbm.at[pl.ds(0, HALF)], sem_o[s]).wait()

    def chunk(c, s, first, fire_next):
        wait_gathers(s)
        if not first:
            wait_store(s)
        add_rows(s)
        fire_store_at(start_c + c, s, sem_o[s])
        if fire_next:
            fire_gathers(c + NSLOT, s)

    for s in range(NSLOT):
        fire_gathers(s, s)
    for j in range(NSLOT):
        chunk(j, j, first=True, fire_next=True)

    def body(m, cy):
        for j in range(NSLOT):
            chunk(NSLOT * m + j, j, first=False, fire_next=True)
        return cy

    lax.fori_loop(1, NTRI - 1, body, 0)
    for j in range(NSLOT):
        chunk(NSLOT * (NTRI - 1) + j, j, first=False, fire_next=False)
    for s in range(NSLOT):
        wait_store(s)

    @pl.when(wid < N_EXTRA)
    def _extra():
        ec = start_c + NCW
        pltpu.sync_copy(ei_hbm.at[pl.ds(ec, 1)], eibuf.at[pl.ds(0, 1)])
        fire_gathers(0, 0)
        wait_gathers(0)
        add_rows(0)
        fire_store_at(ec, 0, sem_o[0])
        wait_store(0)


def kernel(x, edge_index, pos, W, b):
    wcat = jnp.concatenate([W[:D_FEAT, :], W[D_FEAT:, :]], axis=1)
    p, q = pl.pallas_call(
        _tc_tables,
        out_shape=[
            jax.ShapeDtypeStruct((N_NODES, D_EDGE), jnp.float32),
            jax.ShapeDtypeStruct((N_NODES, D_EDGE), jnp.float32),
        ],
    )(x, wcat, b.reshape(1, D_EDGE))
    ei3 = edge_index.reshape(2, N_CHUNKS, G).transpose(1, 0, 2)
    flat = _sc_gather_add(p, q, ei3)
    arr = flat.reshape(2, N_CHUNKS, 8, G)
    return arr.transpose(1, 3, 0, 2).reshape(N_EDGES, D_EDGE)
